# Initial kernel scaffold; baseline (speedup 1.0000x reference)
import jax
import jax.numpy as jnp
from jax.experimental import pallas as pl


def kernel(f, f_aug, label, label_aug):
    raise NotImplementedError("write your pallas kernel here")


# speedup vs baseline: 53.0722x; 53.0722x over previous
import jax
import jax.numpy as jnp
import numpy as np
from jax.experimental import pallas as pl
from jax.experimental.pallas import tpu as pltpu

_NCLS = 4
_KTOP = 3
_BIG = 3.0e7
_NEG = -3.0
_CH = 576


def _proto_kernel(f_ref, fa_ref, lf_ref, la_ref, out1_ref, out2_ref, sim_ref):
    C, N = f_ref.shape[1], f_ref.shape[2]
    fm = f_ref[0]
    fa = fa_ref[0]
    lf = lf_ref[0]
    la = la_ref[0]

    fn = fm / jnp.maximum(jnp.sqrt(jnp.sum(fm * fm, axis=0, keepdims=True)), 1e-12)
    fan = fa / jnp.maximum(jnp.sqrt(jnp.sum(fa * fa, axis=0, keepdims=True)), 1e-12)

    sim_ref[...] = jax.lax.dot_general(
        fn, fan, (((0,), (0,)), ((), ())),
        preferred_element_type=jnp.float32)

    iota_sub = jax.lax.broadcasted_iota(jnp.int32, (N, _CH), 0).astype(jnp.float32)
    iota_lane = jax.lax.broadcasted_iota(jnp.int32, (_CH, N), 1).astype(jnp.float32)

    key_a = iota_sub * 8.0 + lf
    col_parts = []
    for c0 in range(0, N, _CH):
        blk = sim_ref[:, c0:c0 + _CH]
        cmax = jnp.max(blk, axis=0, keepdims=True)
        enc = jnp.min(jnp.where(blk == cmax, key_a, _BIG), axis=0, keepdims=True)
        col_parts.append(enc - jnp.floor(enc * 0.125) * 8.0)
    collabel = jnp.concatenate(col_parts, axis=1)

    key_b = iota_lane * 8.0 + la
    row_parts = []
    for r0 in range(0, N, _CH):
        blk = sim_ref[r0:r0 + _CH, :]
        rmax = jnp.max(blk, axis=1, keepdims=True)
        enc = jnp.min(jnp.where(blk == rmax, key_b, _BIG), axis=1, keepdims=True)
        row_parts.append(enc - jnp.floor(enc * 0.125) * 8.0)
    collabel2 = jnp.concatenate(row_parts, axis=0)

    key_c = iota_lane * 8.0 + collabel
    st1_parts = []
    for r0 in range(0, N, _CH):
        cur = sim_ref[r0:r0 + _CH, :]
        lfc = lf[r0:r0 + _CH]
        any1 = jnp.zeros((_CH, 1), jnp.float32)
        for k in range(_KTOP):
            val = jnp.max(cur, axis=1, keepdims=True)
            enc = jnp.min(jnp.where(cur == val, key_c, _BIG), axis=1, keepdims=True)
            jk = jnp.floor(enc * 0.125)
            labk = enc - jk * 8.0
            any1 = jnp.maximum(any1, (labk == lfc).astype(jnp.float32))
            if k + 1 < _KTOP:
                cur = jnp.where(iota_lane == jk, _NEG, cur)
        st1_parts.append(any1 * (lfc != 4.0).astype(jnp.float32))
    stable1 = jnp.concatenate(st1_parts, axis=0)

    key_d = iota_sub * 8.0 + collabel2
    st2_parts = []
    for c0 in range(0, N, _CH):
        cur = sim_ref[:, c0:c0 + _CH]
        lac = la[:, c0:c0 + _CH]
        any2 = jnp.zeros((1, _CH), jnp.float32)
        for k in range(_KTOP):
            val = jnp.max(cur, axis=0, keepdims=True)
            enc = jnp.min(jnp.where(cur == val, key_d, _BIG), axis=0, keepdims=True)
            ik = jnp.floor(enc * 0.125)
            labk = enc - ik * 8.0
            any2 = jnp.maximum(any2, (labk == lac).astype(jnp.float32))
            if k + 1 < _KTOP:
                cur = jnp.where(iota_sub == ik, _NEG, cur)
        st2_parts.append(any2 * (lac != 4.0).astype(jnp.float32))
    stable2 = jnp.concatenate(st2_parts, axis=1)

    cls1 = [(lf == float(c)).astype(jnp.float32) for c in range(_NCLS)]
    masks1 = jnp.concatenate([m * stable1 for m in cls1] + cls1, axis=1)
    sums1 = jax.lax.dot_general(
        fm, masks1, (((1,), (0,)), ((), ())),
        preferred_element_type=jnp.float32,
        precision=jax.lax.Precision.HIGHEST)
    counts1 = jnp.sum(masks1, axis=0, keepdims=True)
    smean1 = sums1[:, :_NCLS] / jnp.maximum(counts1[:, :_NCLS], 1.0)
    cmean1 = sums1[:, _NCLS:] / jnp.maximum(counts1[:, _NCLS:], 1.0)
    p1 = jnp.where(counts1[:, :_NCLS] > 0.0, smean1,
                   jnp.where(counts1[:, _NCLS:] > 0.0, cmean1, 0.0))
    out1_ref[0] = jnp.concatenate([p1, jnp.zeros((C, _NCLS), jnp.float32)], axis=1)

    cls2 = [(la == float(c)).astype(jnp.float32) for c in range(_NCLS)]
    masks2 = jnp.concatenate([m * stable2 for m in cls2] + cls2, axis=0)
    sums2 = jax.lax.dot_general(
        masks2, fa, (((1,), (1,)), ((), ())),
        preferred_element_type=jnp.float32,
        precision=jax.lax.Precision.HIGHEST)
    counts2 = jnp.sum(masks2, axis=1, keepdims=True)
    smean2 = sums2[:_NCLS] / jnp.maximum(counts2[:_NCLS], 1.0)
    cmean2 = sums2[_NCLS:] / jnp.maximum(counts2[_NCLS:], 1.0)
    p2 = jnp.where(counts2[:_NCLS] > 0.0, smean2,
                   jnp.where(counts2[_NCLS:] > 0.0, cmean2, 0.0))
    out2_ref[0] = jnp.concatenate([p2, jnp.zeros((_NCLS, C), jnp.float32)], axis=0)


def kernel(f, f_aug, label, label_aug):
    B, C, H, W = f.shape
    N = H * W
    fm = f.reshape(B, C, N)
    fam = f_aug.reshape(B, C, N)
    ri = (np.arange(H) * label.shape[-2]) // H
    ci = (np.arange(W) * label.shape[-1]) // W
    lab = label[:, 0][:, ri][:, :, ci].reshape(B, N, 1).astype(jnp.float32)
    laba = label_aug[:, 0][:, ri][:, :, ci].reshape(B, 1, N).astype(jnp.float32)

    out1, out2 = pl.pallas_call(
        _proto_kernel,
        grid=(B,),
        in_specs=[
            pl.BlockSpec((1, C, N), lambda b: (b, 0, 0)),
            pl.BlockSpec((1, C, N), lambda b: (b, 0, 0)),
            pl.BlockSpec((1, N, 1), lambda b: (b, 0, 0)),
            pl.BlockSpec((1, 1, N), lambda b: (b, 0, 0)),
        ],
        out_specs=[
            pl.BlockSpec((1, C, 2 * _NCLS), lambda b: (b, 0, 0)),
            pl.BlockSpec((1, 2 * _NCLS, C), lambda b: (b, 0, 0)),
        ],
        out_shape=[
            jax.ShapeDtypeStruct((B, C, 2 * _NCLS), jnp.float32),
            jax.ShapeDtypeStruct((B, 2 * _NCLS, C), jnp.float32),
        ],
        scratch_shapes=[pltpu.VMEM((N, N), jnp.float32)],
    )(fm, fam, lab, laba)
    proto = jnp.transpose(out1[:, :, :_NCLS], (0, 2, 1))
    proto_aug = out2[:, :_NCLS, :]
    return (proto, proto_aug)


# speedup vs baseline: 53.0881x; 1.0003x over previous
import jax
import jax.numpy as jnp
import numpy as np
from jax.experimental import pallas as pl
from jax.experimental.pallas import tpu as pltpu

_NCLS = 4
_KTOP = 3
_BIG = 3.0e7
_NEG = -3.0
_CH = 576


def _proto_kernel(f_ref, fa_ref, lf_ref, la_ref, out1_ref, out2_ref, sim_ref):
    C, N = f_ref.shape[1], f_ref.shape[2]
    fm = f_ref[0]
    fa = fa_ref[0]
    lf = lf_ref[0]
    la = la_ref[0]

    fn = fm / jnp.maximum(jnp.sqrt(jnp.sum(fm * fm, axis=0, keepdims=True)), 1e-12)
    fan = fa / jnp.maximum(jnp.sqrt(jnp.sum(fa * fa, axis=0, keepdims=True)), 1e-12)

    sim_ref[...] = jax.lax.dot_general(
        fn, fan, (((0,), (0,)), ((), ())),
        preferred_element_type=jnp.float32)

    iota_sub = jax.lax.broadcasted_iota(jnp.int32, (N, _CH), 0).astype(jnp.float32)
    iota_lane = jax.lax.broadcasted_iota(jnp.int32, (_CH, N), 1).astype(jnp.float32)

    key_a = iota_sub * 8.0 + lf
    col_parts = []
    for c0 in range(0, N, _CH):
        blk = sim_ref[:, c0:c0 + _CH]
        cmax = jnp.max(blk, axis=0, keepdims=True)
        enc = jnp.min(jnp.where(blk == cmax, key_a, _BIG), axis=0, keepdims=True)
        col_parts.append(enc - jnp.floor(enc * 0.125) * 8.0)
    collabel = jnp.concatenate(col_parts, axis=1)

    key_b = iota_lane * 8.0 + la
    row_parts = []
    for r0 in range(0, N, _CH):
        blk = sim_ref[r0:r0 + _CH, :]
        rmax = jnp.max(blk, axis=1, keepdims=True)
        enc = jnp.min(jnp.where(blk == rmax, key_b, _BIG), axis=1, keepdims=True)
        row_parts.append(enc - jnp.floor(enc * 0.125) * 8.0)
    collabel2 = jnp.concatenate(row_parts, axis=0)

    key_c = iota_lane * 8.0 + collabel
    st1_parts = []
    for r0 in range(0, N, _CH):
        cur = sim_ref[r0:r0 + _CH, :]
        lfc = lf[r0:r0 + _CH]
        any1 = jnp.zeros((_CH, 1), jnp.float32)
        for k in range(_KTOP):
            val = jnp.max(cur, axis=1, keepdims=True)
            enc = jnp.min(jnp.where(cur == val, key_c, _BIG), axis=1, keepdims=True)
            jk = jnp.floor(enc * 0.125)
            labk = enc - jk * 8.0
            any1 = jnp.maximum(any1, (labk == lfc).astype(jnp.float32))
            if k + 1 < _KTOP:
                cur = jnp.where(iota_lane == jk, _NEG, cur)
        st1_parts.append(any1 * (lfc != 4.0).astype(jnp.float32))
    stable1 = jnp.concatenate(st1_parts, axis=0)

    key_d = iota_sub * 8.0 + collabel2
    st2_parts = []
    for c0 in range(0, N, _CH):
        cur = sim_ref[:, c0:c0 + _CH]
        lac = la[:, c0:c0 + _CH]
        any2 = jnp.zeros((1, _CH), jnp.float32)
        for k in range(_KTOP):
            val = jnp.max(cur, axis=0, keepdims=True)
            enc = jnp.min(jnp.where(cur == val, key_d, _BIG), axis=0, keepdims=True)
            ik = jnp.floor(enc * 0.125)
            labk = enc - ik * 8.0
            any2 = jnp.maximum(any2, (labk == lac).astype(jnp.float32))
            if k + 1 < _KTOP:
                cur = jnp.where(iota_sub == ik, _NEG, cur)
        st2_parts.append(any2 * (lac != 4.0).astype(jnp.float32))
    stable2 = jnp.concatenate(st2_parts, axis=1)

    cls1 = [(lf == float(c)).astype(jnp.float32) for c in range(_NCLS)]
    masks1 = jnp.concatenate([m * stable1 for m in cls1] + cls1, axis=1)
    sums1 = jax.lax.dot_general(
        fm, masks1, (((1,), (0,)), ((), ())),
        preferred_element_type=jnp.float32,
        precision=jax.lax.Precision.HIGHEST)
    counts1 = jnp.sum(masks1, axis=0, keepdims=True)
    smean1 = sums1[:, :_NCLS] / jnp.maximum(counts1[:, :_NCLS], 1.0)
    cmean1 = sums1[:, _NCLS:] / jnp.maximum(counts1[:, _NCLS:], 1.0)
    p1 = jnp.where(counts1[:, :_NCLS] > 0.0, smean1,
                   jnp.where(counts1[:, _NCLS:] > 0.0, cmean1, 0.0))
    out1_ref[0] = jnp.concatenate([p1, jnp.zeros((C, _NCLS), jnp.float32)], axis=1)

    cls2 = [(la == float(c)).astype(jnp.float32) for c in range(_NCLS)]
    masks2 = jnp.concatenate([m * stable2 for m in cls2] + cls2, axis=0)
    sums2 = jax.lax.dot_general(
        masks2, fa, (((1,), (1,)), ((), ())),
        preferred_element_type=jnp.float32,
        precision=jax.lax.Precision.HIGHEST)
    counts2 = jnp.sum(masks2, axis=1, keepdims=True)
    smean2 = sums2[:_NCLS] / jnp.maximum(counts2[:_NCLS], 1.0)
    cmean2 = sums2[_NCLS:] / jnp.maximum(counts2[_NCLS:], 1.0)
    p2 = jnp.where(counts2[:_NCLS] > 0.0, smean2,
                   jnp.where(counts2[_NCLS:] > 0.0, cmean2, 0.0))
    out2_ref[0] = jnp.concatenate([p2, jnp.zeros((_NCLS, C), jnp.float32)], axis=0)


def kernel(f, f_aug, label, label_aug):
    B, C, H, W = f.shape
    N = H * W
    fm = f.reshape(B, C, N)
    fam = f_aug.reshape(B, C, N)
    ri = (np.arange(H) * label.shape[-2]) // H
    ci = (np.arange(W) * label.shape[-1]) // W
    lab = label[:, 0][:, ri][:, :, ci].reshape(B, N, 1).astype(jnp.float32)
    laba = label_aug[:, 0][:, ri][:, :, ci].reshape(B, 1, N).astype(jnp.float32)

    out1, out2 = pl.pallas_call(
        _proto_kernel,
        grid=(B,),
        in_specs=[
            pl.BlockSpec((1, C, N), lambda b: (b, 0, 0)),
            pl.BlockSpec((1, C, N), lambda b: (b, 0, 0)),
            pl.BlockSpec((1, N, 1), lambda b: (b, 0, 0)),
            pl.BlockSpec((1, 1, N), lambda b: (b, 0, 0)),
        ],
        out_specs=[
            pl.BlockSpec((1, C, 2 * _NCLS), lambda b: (b, 0, 0)),
            pl.BlockSpec((1, 2 * _NCLS, C), lambda b: (b, 0, 0)),
        ],
        out_shape=[
            jax.ShapeDtypeStruct((B, C, 2 * _NCLS), jnp.float32),
            jax.ShapeDtypeStruct((B, 2 * _NCLS, C), jnp.float32),
        ],
        scratch_shapes=[pltpu.VMEM((N, N), jnp.float32)],
        compiler_params=pltpu.CompilerParams(
            dimension_semantics=("parallel",)),
    )(fm, fam, lab, laba)
    proto = jnp.transpose(out1[:, :, :_NCLS], (0, 2, 1))
    proto_aug = out2[:, :_NCLS, :]
    return (proto, proto_aug)
